# Initial kernel scaffold; baseline (speedup 1.0000x reference)
import jax
import jax.numpy as jnp
from jax.experimental import pallas as pl


def kernel(x, coeff1, bias1, coeff2, bias2, W, b):
    raise NotImplementedError("write your pallas kernel here")


# speedup vs baseline: 168.1030x; 168.1030x over previous
import functools

import jax
import jax.numpy as jnp
from jax.experimental import pallas as pl
from jax.experimental.pallas import tpu as pltpu

B = 16384
IN_DIM = 100
F_PAD = 128
HIDDEN = 64
GRID = 16
X_MIN, X_MAX = -3.0, 3.0
INV_STEP = (GRID - 1) / (X_MAX - X_MIN)
BB = 512


def _interp_weights(v):
    p = (v - X_MIN) * INV_STEP
    idx = jnp.clip(jnp.ceil(p), 1.0, GRID - 1.0)
    t = p - idx + 1.0
    return idx, t, 1.0 - t


def _kan_matmuls(idx, t, w0, c_ref):
    acc = jnp.zeros((idx.shape[0], HIDDEN), dtype=jnp.float32)
    for q in range(GRID):
        if q == 0:
            a = jnp.where(idx == 1.0, w0, 0.0)
        elif q == GRID - 1:
            a = jnp.where(idx == float(q), t, 0.0)
        else:
            a = jnp.where(idx == float(q), t, jnp.where(idx == float(q + 1), w0, 0.0))
        acc += jnp.dot(
            a,
            c_ref[q],
            preferred_element_type=jnp.float32,
            precision=jax.lax.Precision.HIGHEST,
        )
    return acc


def _fwd_kernel(x_ref, c1_ref, b1_ref, c2_ref, b2_ref, wt_ref, bout_ref, out_ref):
    x = x_ref[...]
    idx1, t1, w01 = _interp_weights(x)
    h = _kan_matmuls(idx1, t1, w01, c1_ref) + b1_ref[...]
    h = jnp.maximum(h, 0.0)

    idx2, t2, w02 = _interp_weights(h)
    h2 = _kan_matmuls(idx2, t2, w02, c2_ref) + b2_ref[...]
    h2 = jnp.maximum(h2, 0.0)

    out_ref[...] = (
        jnp.sum(h2 * wt_ref[...], axis=1, keepdims=True) + bout_ref[...]
    )


@jax.jit
def kernel(x, coeff1, bias1, coeff2, bias2, W, b):
    c1 = jnp.pad(
        jnp.transpose(coeff1, (1, 0, 2)), ((0, 0), (0, F_PAD - IN_DIM), (0, 0))
    )
    c2 = jnp.transpose(coeff2, (1, 0, 2))
    x_p = jnp.pad(x, ((0, 0), (0, F_PAD - IN_DIM)))
    b1 = bias1.reshape(1, HIDDEN)
    b2 = bias2.reshape(1, HIDDEN)
    wt = W.reshape(1, HIDDEN)
    bout = b.reshape(1, 1)

    n_blocks = B // BB
    out = pl.pallas_call(
        _fwd_kernel,
        grid=(n_blocks,),
        in_specs=[
            pl.BlockSpec((BB, F_PAD), lambda i: (i, 0)),
            pl.BlockSpec((GRID, F_PAD, HIDDEN), lambda i: (0, 0, 0)),
            pl.BlockSpec((1, HIDDEN), lambda i: (0, 0)),
            pl.BlockSpec((GRID, HIDDEN, HIDDEN), lambda i: (0, 0, 0)),
            pl.BlockSpec((1, HIDDEN), lambda i: (0, 0)),
            pl.BlockSpec((1, HIDDEN), lambda i: (0, 0)),
            pl.BlockSpec((1, 1), lambda i: (0, 0)),
        ],
        out_specs=pl.BlockSpec((BB, 1), lambda i: (i, 0)),
        out_shape=jax.ShapeDtypeStruct((B, 1), jnp.float32),
        compiler_params=pltpu.CompilerParams(
            dimension_semantics=("arbitrary",),
        ),
    )(x_p, c1, b1, c2, b2, wt, bout)
    return out[:, 0]


# speedup vs baseline: 754.2808x; 4.4870x over previous
import jax
import jax.numpy as jnp
from jax.experimental import pallas as pl
from jax.experimental.pallas import tpu as pltpu

B = 16384
IN_DIM = 100
F_PAD = 128
HIDDEN = 64
GRID = 16
X_MIN, X_MAX = -3.0, 3.0
INV_STEP = (GRID - 1) / (X_MAX - X_MIN)
BB = 512
Q2_LO = 7


def _a_piece(p, q):
    if q == 0:
        return jnp.maximum(1.0 - p, 0.0)
    if q == 1:
        return jnp.minimum(p, jnp.maximum(2.0 - p, 0.0))
    if q == GRID - 2:
        return jnp.minimum(jnp.maximum(p - (GRID - 3.0), 0.0), (GRID - 1.0) - p)
    if q == GRID - 1:
        return jnp.maximum(p - (GRID - 2.0), 0.0)
    return jnp.maximum(1.0 - jnp.abs(p - float(q)), 0.0)


def _split_bf16(a):
    au = jax.lax.bitcast_convert_type(a, jnp.uint32)
    ah = jax.lax.bitcast_convert_type(
        au & jnp.uint32(0xFFFF0000), jnp.float32
    )
    return ah, a - ah


def _round_bf16(a):
    au = jax.lax.bitcast_convert_type(a, jnp.uint32)
    rounded = (au + jnp.uint32(0x7FFF) + ((au >> 16) & jnp.uint32(1))) & jnp.uint32(
        0xFFFF0000
    )
    return jax.lax.bitcast_convert_type(rounded, jnp.float32)


def _kan_layer(v, c_cat_ref, c_hi_ref, b_ref, q_lo):
    p = (v - X_MIN) * INV_STEP
    a = jnp.concatenate([_a_piece(p, q) for q in range(q_lo, GRID)], axis=1)
    a_hi, a_lo = _split_bf16(a)
    r = jnp.dot(a_hi, c_cat_ref[...], preferred_element_type=jnp.float32)
    r_lo = jnp.dot(a_lo, c_hi_ref[...], preferred_element_type=jnp.float32)
    return r[:, :HIDDEN] + r[:, HIDDEN:] + r_lo + b_ref[...]


def _fwd_kernel(
    x_ref, c1cat_ref, c1hi_ref, b1_ref, c2cat_ref, c2hi_ref, b2_ref,
    wt_ref, bout_ref, out_ref,
):
    x = x_ref[...]
    h = jnp.maximum(_kan_layer(x, c1cat_ref, c1hi_ref, b1_ref, 0), 0.0)
    h2 = jnp.maximum(_kan_layer(h, c2cat_ref, c2hi_ref, b2_ref, Q2_LO), 0.0)
    out_ref[...] = (
        jnp.sum(_round_bf16(h2) * wt_ref[...], axis=1, keepdims=True)
        + bout_ref[...]
    )


@jax.jit
def kernel(x, coeff1, bias1, coeff2, bias2, W, b):
    c1 = jnp.pad(
        jnp.transpose(coeff1, (1, 0, 2)), ((0, 0), (0, F_PAD - IN_DIM), (0, 0))
    ).reshape(GRID * F_PAD, HIDDEN)
    c2 = jnp.transpose(coeff2, (1, 0, 2))[Q2_LO:].reshape(
        (GRID - Q2_LO) * HIDDEN, HIDDEN
    )
    c1_hi, c1_lo = _split_bf16(c1)
    c2_hi, c2_lo = _split_bf16(c2)
    c1_cat = jnp.concatenate([c1_hi, _split_bf16(c1_lo)[0]], axis=1)
    c2_cat = jnp.concatenate([c2_hi, _split_bf16(c2_lo)[0]], axis=1)
    x_p = jnp.pad(x, ((0, 0), (0, F_PAD - IN_DIM)))
    b1 = bias1.reshape(1, HIDDEN)
    b2 = bias2.reshape(1, HIDDEN)
    wt = _round_bf16(W.reshape(1, HIDDEN))
    bout = b.reshape(1, 1)

    k1 = GRID * F_PAD
    k2 = (GRID - Q2_LO) * HIDDEN
    n_blocks = B // BB
    out = pl.pallas_call(
        _fwd_kernel,
        grid=(n_blocks,),
        in_specs=[
            pl.BlockSpec((BB, F_PAD), lambda i: (i, 0)),
            pl.BlockSpec((k1, 2 * HIDDEN), lambda i: (0, 0)),
            pl.BlockSpec((k1, HIDDEN), lambda i: (0, 0)),
            pl.BlockSpec((1, HIDDEN), lambda i: (0, 0)),
            pl.BlockSpec((k2, 2 * HIDDEN), lambda i: (0, 0)),
            pl.BlockSpec((k2, HIDDEN), lambda i: (0, 0)),
            pl.BlockSpec((1, HIDDEN), lambda i: (0, 0)),
            pl.BlockSpec((1, HIDDEN), lambda i: (0, 0)),
            pl.BlockSpec((1, 1), lambda i: (0, 0)),
        ],
        out_specs=pl.BlockSpec((BB, 1), lambda i: (i, 0)),
        out_shape=jax.ShapeDtypeStruct((B, 1), jnp.float32),
        compiler_params=pltpu.CompilerParams(
            dimension_semantics=("arbitrary",),
        ),
    )(x_p, c1_cat, c1_hi, b1, c2_cat, c2_hi, b2, wt, bout)
    return out[:, 0]


# speedup vs baseline: 888.2169x; 1.1776x over previous
import jax
import jax.numpy as jnp
from jax.experimental import pallas as pl
from jax.experimental.pallas import tpu as pltpu

B = 16384
IN_DIM = 100
F_PAD = 128
HIDDEN = 64
GRID = 16
X_MIN, X_MAX = -3.0, 3.0
INV_STEP = (GRID - 1) / (X_MAX - X_MIN)
BB = 512
Q2_LO = 7


def _a_piece(p, q):
    if q == 0:
        return jnp.maximum(1.0 - p, 0.0)
    if q == 1:
        return jnp.minimum(p, jnp.maximum(2.0 - p, 0.0))
    if q == GRID - 2:
        return jnp.minimum(jnp.maximum(p - (GRID - 3.0), 0.0), (GRID - 1.0) - p)
    if q == GRID - 1:
        return jnp.maximum(p - (GRID - 2.0), 0.0)
    return jnp.maximum(1.0 - jnp.abs(p - float(q)), 0.0)


def _split_bf16(a):
    au = jax.lax.bitcast_convert_type(a, jnp.uint32)
    ah = jax.lax.bitcast_convert_type(au & jnp.uint32(0xFFFF0000), jnp.float32)
    return ah, a - ah


def _round_bf16(a):
    au = jax.lax.bitcast_convert_type(a, jnp.uint32)
    rounded = (au + jnp.uint32(0x7FFF) + ((au >> 16) & jnp.uint32(1))) & jnp.uint32(
        0xFFFF0000
    )
    return jax.lax.bitcast_convert_type(rounded, jnp.float32)


def _kan_layer(v, c_cat_ref, c_hi_ref, b_ref, q_lo):
    p = (v - X_MIN) * INV_STEP
    a = jnp.concatenate([_a_piece(p, q) for q in range(q_lo, GRID)], axis=1)
    a_hi, a_lo = _split_bf16(a)
    r = jnp.dot(
        a_hi.astype(jnp.bfloat16),
        c_cat_ref[...],
        preferred_element_type=jnp.float32,
    )
    r_lo = jnp.dot(
        a_lo.astype(jnp.bfloat16),
        c_hi_ref[...],
        preferred_element_type=jnp.float32,
    )
    return r[:, :HIDDEN] + r[:, HIDDEN:] + r_lo + b_ref[...]


def _fwd_kernel(
    x_ref, c1cat_ref, c1hi_ref, b1_ref, c2cat_ref, c2hi_ref, b2_ref,
    wt_ref, bout_ref, out_ref,
):
    x = jnp.pad(x_ref[...], ((0, 0), (0, F_PAD - IN_DIM)))
    h = jnp.maximum(_kan_layer(x, c1cat_ref, c1hi_ref, b1_ref, 0), 0.0)
    h2 = jnp.maximum(_kan_layer(h, c2cat_ref, c2hi_ref, b2_ref, Q2_LO), 0.0)
    out_ref[...] = (
        jnp.sum(_round_bf16(h2) * wt_ref[...], axis=1, keepdims=True)
        + bout_ref[...]
    )


@jax.jit
def kernel(x, coeff1, bias1, coeff2, bias2, W, b):
    c1 = jnp.pad(
        jnp.transpose(coeff1, (1, 0, 2)), ((0, 0), (0, F_PAD - IN_DIM), (0, 0))
    ).reshape(GRID * F_PAD, HIDDEN)
    c2 = jnp.transpose(coeff2, (1, 0, 2))[Q2_LO:].reshape(
        (GRID - Q2_LO) * HIDDEN, HIDDEN
    )
    c1_hi, c1_lo = _split_bf16(c1)
    c2_hi, c2_lo = _split_bf16(c2)
    c1_cat = jnp.concatenate([c1_hi, c1_lo], axis=1).astype(jnp.bfloat16)
    c2_cat = jnp.concatenate([c2_hi, c2_lo], axis=1).astype(jnp.bfloat16)
    c1_hi = c1_hi.astype(jnp.bfloat16)
    c2_hi = c2_hi.astype(jnp.bfloat16)
    b1 = bias1.reshape(1, HIDDEN)
    b2 = bias2.reshape(1, HIDDEN)
    wt = _round_bf16(W.reshape(1, HIDDEN))
    bout = b.reshape(1, 1)

    k1 = GRID * F_PAD
    k2 = (GRID - Q2_LO) * HIDDEN
    n_blocks = B // BB
    out = pl.pallas_call(
        _fwd_kernel,
        grid=(n_blocks,),
        in_specs=[
            pl.BlockSpec((BB, IN_DIM), lambda i: (i, 0)),
            pl.BlockSpec((k1, 2 * HIDDEN), lambda i: (0, 0)),
            pl.BlockSpec((k1, HIDDEN), lambda i: (0, 0)),
            pl.BlockSpec((1, HIDDEN), lambda i: (0, 0)),
            pl.BlockSpec((k2, 2 * HIDDEN), lambda i: (0, 0)),
            pl.BlockSpec((k2, HIDDEN), lambda i: (0, 0)),
            pl.BlockSpec((1, HIDDEN), lambda i: (0, 0)),
            pl.BlockSpec((1, HIDDEN), lambda i: (0, 0)),
            pl.BlockSpec((1, 1), lambda i: (0, 0)),
        ],
        out_specs=pl.BlockSpec((BB, 1), lambda i: (i, 0)),
        out_shape=jax.ShapeDtypeStruct((B, 1), jnp.float32),
        compiler_params=pltpu.CompilerParams(
            dimension_semantics=("arbitrary",),
        ),
    )(x, c1_cat, c1_hi, b1, c2_cat, c2_hi, b2, wt, bout)
    return out[:, 0]


# speedup vs baseline: 960.5393x; 1.0814x over previous
import jax
import jax.numpy as jnp
from jax.experimental import pallas as pl
from jax.experimental.pallas import tpu as pltpu

B = 16384
IN_DIM = 100
F_PAD = 128
HIDDEN = 64
GRID = 16
X_MIN, X_MAX = -3.0, 3.0
INV_STEP = (GRID - 1) / (X_MAX - X_MIN)
BB = 1024
Q2_LO = 7


def _a_piece(p, q):
    if q == 0:
        return jnp.maximum(1.0 - p, 0.0)
    if q == 1:
        return jnp.minimum(p, jnp.maximum(2.0 - p, 0.0))
    if q == GRID - 2:
        return jnp.minimum(jnp.maximum(p - (GRID - 3.0), 0.0), (GRID - 1.0) - p)
    if q == GRID - 1:
        return jnp.maximum(p - (GRID - 2.0), 0.0)
    return jnp.maximum(1.0 - jnp.abs(p - float(q)), 0.0)


def _split_bf16(a):
    au = jax.lax.bitcast_convert_type(a, jnp.uint32)
    ah = jax.lax.bitcast_convert_type(au & jnp.uint32(0xFFFF0000), jnp.float32)
    return ah, a - ah


def _round_bf16(a):
    au = jax.lax.bitcast_convert_type(a, jnp.uint32)
    rounded = (au + jnp.uint32(0x7FFF) + ((au >> 16) & jnp.uint32(1))) & jnp.uint32(
        0xFFFF0000
    )
    return jax.lax.bitcast_convert_type(rounded, jnp.float32)


def _kan_layer(v, c_cat_ref, c_hi_ref, b_ref, q_lo):
    p = (v - X_MIN) * INV_STEP
    a = jnp.concatenate([_a_piece(p, q) for q in range(q_lo, GRID)], axis=1)
    a_hi, a_lo = _split_bf16(a)
    r = jnp.dot(
        a_hi.astype(jnp.bfloat16),
        c_cat_ref[...],
        preferred_element_type=jnp.float32,
    )
    r_lo = jnp.dot(
        a_lo.astype(jnp.bfloat16),
        c_hi_ref[...],
        preferred_element_type=jnp.float32,
    )
    return r[:, :HIDDEN] + r[:, HIDDEN:] + r_lo + b_ref[...]


def _fwd_kernel(
    x_ref, c1cat_ref, c1hi_ref, b1_ref, c2cat_ref, c2hi_ref, b2_ref,
    wt_ref, bout_ref, out_ref,
):
    x = jnp.pad(x_ref[...], ((0, 0), (0, F_PAD - IN_DIM)))
    h = jnp.maximum(_kan_layer(x, c1cat_ref, c1hi_ref, b1_ref, 0), 0.0)
    h2 = jnp.maximum(_kan_layer(h, c2cat_ref, c2hi_ref, b2_ref, Q2_LO), 0.0)
    out_ref[...] = (
        jnp.sum(_round_bf16(h2) * wt_ref[...], axis=1, keepdims=True)
        + bout_ref[...]
    )


@jax.jit
def kernel(x, coeff1, bias1, coeff2, bias2, W, b):
    c1 = jnp.pad(
        jnp.transpose(coeff1, (1, 0, 2)), ((0, 0), (0, F_PAD - IN_DIM), (0, 0))
    ).reshape(GRID * F_PAD, HIDDEN)
    c2 = jnp.transpose(coeff2, (1, 0, 2))[Q2_LO:].reshape(
        (GRID - Q2_LO) * HIDDEN, HIDDEN
    )
    c1_hi, c1_lo = _split_bf16(c1)
    c2_hi, c2_lo = _split_bf16(c2)
    c1_cat = jnp.concatenate([c1_hi, c1_lo], axis=1).astype(jnp.bfloat16)
    c2_cat = jnp.concatenate([c2_hi, c2_lo], axis=1).astype(jnp.bfloat16)
    c1_hi = c1_hi.astype(jnp.bfloat16)
    c2_hi = c2_hi.astype(jnp.bfloat16)
    b1 = bias1.reshape(1, HIDDEN)
    b2 = bias2.reshape(1, HIDDEN)
    wt = _round_bf16(W.reshape(1, HIDDEN))
    bout = b.reshape(1, 1)

    k1 = GRID * F_PAD
    k2 = (GRID - Q2_LO) * HIDDEN
    n_blocks = B // BB
    out = pl.pallas_call(
        _fwd_kernel,
        grid=(n_blocks,),
        in_specs=[
            pl.BlockSpec((BB, IN_DIM), lambda i: (i, 0)),
            pl.BlockSpec((k1, 2 * HIDDEN), lambda i: (0, 0)),
            pl.BlockSpec((k1, HIDDEN), lambda i: (0, 0)),
            pl.BlockSpec((1, HIDDEN), lambda i: (0, 0)),
            pl.BlockSpec((k2, 2 * HIDDEN), lambda i: (0, 0)),
            pl.BlockSpec((k2, HIDDEN), lambda i: (0, 0)),
            pl.BlockSpec((1, HIDDEN), lambda i: (0, 0)),
            pl.BlockSpec((1, HIDDEN), lambda i: (0, 0)),
            pl.BlockSpec((1, 1), lambda i: (0, 0)),
        ],
        out_specs=pl.BlockSpec((BB, 1), lambda i: (i, 0)),
        out_shape=jax.ShapeDtypeStruct((B, 1), jnp.float32),
        compiler_params=pltpu.CompilerParams(
            dimension_semantics=("arbitrary",),
        ),
    )(x, c1_cat, c1_hi, b1, c2_cat, c2_hi, b2, wt, bout)
    return out[:, 0]
